# baseline (speedup 1.0000x reference)
import functools
import math

import jax
import jax.numpy as jnp
from jax import lax
from jax.experimental import pallas as pl
from jax.experimental.pallas import tpu as pltpu
from jax.experimental.pallas import tpu_sc as plsc

_N_NODES = 10000
_N_EDGES = 320000
_NUM_GRAPHS = 64
_LOG_STD_MIN = -20.0
_LOG_STD_MAX = 2.0

_EC = 2560
_EG = _N_EDGES // _EC

_NC = 2
_NS = 16
_NW = _NC * _NS
_EPW = _N_EDGES // _NW
_CH = 80
_GRP = 5
_GSZ = _CH * _GRP
_NGRP = _EPW // _GSZ
_ACC = 10240
_TPS = _ACC // _NS

_SC_MESH = plsc.VectorSubcoreMesh(
    core_axis_name="c", subcore_axis_name="s", num_cores=_NC, num_subcores=_NS)
_SC_PARAMS = pltpu.CompilerParams(use_tc_tiling_on_sc=False)


def _sc_gather_body(gs_hbm, gd_hbm, row_hbm, col_hbm, ga_hbm, gb_hbm,
                    idx_a, idx_b, buf_a, buf_b, sem):
    wid = lax.axis_index("s") * _NC + lax.axis_index("c")
    base = wid * _EPW

    def grp(o, carry):
        gbase = pl.multiple_of(base + o * _GSZ, _GSZ)
        pltpu.sync_copy(row_hbm.at[pl.ds(gbase, _GSZ)], idx_a)
        pltpu.sync_copy(col_hbm.at[pl.ds(gbase, _GSZ)], idx_b)
        hs = []
        for j in range(_GRP):
            off = j * _CH
            hs.append(pltpu.async_copy(
                gs_hbm.at[idx_a.at[pl.ds(off, _CH)]],
                buf_a.at[pl.ds(off, _CH)], sem))
            hs.append(pltpu.async_copy(
                gd_hbm.at[idx_b.at[pl.ds(off, _CH)]],
                buf_b.at[pl.ds(off, _CH)], sem))
        for h in hs:
            h.wait()
        pltpu.sync_copy(buf_a, ga_hbm.at[pl.ds(gbase, _GSZ)])
        pltpu.sync_copy(buf_b, gb_hbm.at[pl.ds(gbase, _GSZ)])
        return carry

    lax.fori_loop(0, _NGRP, grp, 0)


def _sc_gather(gs, gd, row, col):
    return pl.kernel(
        _sc_gather_body,
        out_type=[_f32((_N_EDGES, 64)), _f32((_N_EDGES, 64))],
        mesh=_SC_MESH,
        scratch_types=[
            pltpu.VMEM((_GSZ,), jnp.int32),
            pltpu.VMEM((_GSZ,), jnp.int32),
            pltpu.VMEM((_GSZ, 64), jnp.float32),
            pltpu.VMEM((_GSZ, 64), jnp.float32),
            pltpu.SemaphoreType.DMA,
        ],
        compiler_params=_SC_PARAMS,
    )(gs, gd, row, col)


def _sc_scatter_body(with_cnt, ea2_hbm, col_hbm, z32_hbm, z16_hbm, ones_hbm,
                     out_hbm, cnt_hbm,
                     idx2, val_buf, stage32, stage16, ones_v,
                     acc, acc_cnt, sem, sem2):
    cid = lax.axis_index("c")
    sid = lax.axis_index("s")
    wid = sid * _NC + cid
    base = wid * _EPW
    trow = pl.multiple_of(sid * _TPS, _TPS)

    pltpu.sync_copy(z32_hbm.at[pl.ds(trow, _TPS)], stage32)
    pltpu.sync_copy(stage32, acc.at[pl.ds(trow, _TPS)])
    if with_cnt:
        pltpu.sync_copy(z16_hbm.at[pl.ds(trow, _TPS)], stage16)
        pltpu.sync_copy(stage16, acc_cnt.at[pl.ds(trow, _TPS)])
        pltpu.sync_copy(ones_hbm, ones_v)
    plsc.subcore_barrier()

    def grp(o, carry):
        gbase = pl.multiple_of(base + o * _GSZ, _GSZ)
        hs = [pltpu.async_copy(ea2_hbm.at[pl.ds(gbase, _GSZ)], val_buf, sem)]
        for j in range(_GRP):
            hs.append(pltpu.async_copy(
                col_hbm.at[pl.ds(gbase + j * _CH, _CH)], idx2.at[j], sem))
        for h in hs:
            h.wait()
        hs = []
        for j in range(_GRP):
            hs.append(pltpu.async_copy(
                val_buf.at[pl.ds(j * _CH, _CH)], acc.at[idx2.at[j]], sem2,
                add=True))
            if with_cnt:
                hs.append(pltpu.async_copy(
                    ones_v, acc_cnt.at[idx2.at[j]], sem2, add=True))
        for h in hs:
            h.wait()
        return carry

    lax.fori_loop(0, _NGRP, grp, 0)
    plsc.subcore_barrier()

    orow = pl.multiple_of(cid * _ACC + sid * _TPS, _TPS)
    pltpu.sync_copy(acc.at[pl.ds(trow, _TPS)], stage32)
    pltpu.sync_copy(stage32, out_hbm.at[pl.ds(orow, _TPS)])
    if with_cnt:
        pltpu.sync_copy(acc_cnt.at[pl.ds(trow, _TPS)], stage16)
        pltpu.sync_copy(stage16, cnt_hbm.at[pl.ds(orow, _TPS)])


def _sc_scatter(ea2, col, with_cnt):
    z32 = jnp.zeros((_ACC, 32), jnp.float32)
    z16 = jnp.zeros((_ACC, 16), jnp.float32)
    ones = jnp.ones((_CH, 16), jnp.float32)
    out_types = [_f32((_NC * _ACC, 32)), _f32((_NC * _ACC, 16))]
    res = pl.kernel(
        functools.partial(_sc_scatter_body, with_cnt),
        out_type=out_types,
        mesh=_SC_MESH,
        scratch_types=[
            pltpu.VMEM((_GRP, _CH), jnp.int32),
            pltpu.VMEM((_GSZ, 32), jnp.float32),
            pltpu.VMEM((_TPS, 32), jnp.float32),
            pltpu.VMEM((_TPS, 16), jnp.float32),
            pltpu.VMEM((_CH, 16), jnp.float32),
            pltpu.VMEM_SHARED((_ACC, 32), jnp.float32),
            pltpu.VMEM_SHARED((_ACC, 16), jnp.float32),
            pltpu.SemaphoreType.DMA,
            pltpu.SemaphoreType.DMA,
        ],
        compiler_params=_SC_PARAMS,
    )(ea2, col, z32, z16, ones)
    return res


def _edge_specs(dims_in, dims_out, weights):
    in_specs = [pl.BlockSpec((_EC, d), lambda i: (i, 0)) for d in dims_in]
    in_specs += [pl.BlockSpec(w.shape, lambda i: (0, 0)) for w in weights]
    out_specs = [pl.BlockSpec((_EC, d), lambda i: (i, 0)) for d in dims_out]
    return in_specs, out_specs


def _ln(v, g, b):
    m = jnp.mean(v, axis=-1, keepdims=True)
    c = v - m
    var = jnp.mean(c * c, axis=-1, keepdims=True)
    return c * jax.lax.rsqrt(var + 1e-5) * g + b


def _node0_body(x_ref, wx_ref, bx_ref, w1s_ref, w1d_ref, h_ref, gs_ref, gd_ref):
    h = jnp.maximum(
        jnp.dot(x_ref[...], wx_ref[...], preferred_element_type=jnp.float32)
        + bx_ref[...], 0.0)
    h_ref[...] = h
    gs_ref[...] = jnp.dot(h, w1s_ref[...], preferred_element_type=jnp.float32)
    gd_ref[...] = jnp.dot(h, w1d_ref[...], preferred_element_type=jnp.float32)


def _edge0_body(eattr_ref, we_ref, be_ref, w1e_ref, b1_ref, ea_ref, eaw_ref):
    ea = jnp.maximum(
        jnp.dot(eattr_ref[...], we_ref[...], preferred_element_type=jnp.float32)
        + be_ref[...], 0.0)
    ea_ref[...] = ea
    eaw_ref[...] = jnp.dot(ea, w1e_ref[...], preferred_element_type=jnp.float32) + b1_ref[...]


def _edge_mid_body(ga_ref, gb_ref, eaw_ref, ea_ref,
                   ew2_ref, eb2_ref, g_ref, b_ref, w1e_ref, b1_ref,
                   ea2_ref, ealn_ref, eawn_ref):
    hid = jnp.maximum(ga_ref[...] + gb_ref[...] + eaw_ref[...], 0.0)
    ea2 = ea_ref[...] + jnp.dot(hid, ew2_ref[...], preferred_element_type=jnp.float32) + eb2_ref[...]
    ea2_ref[...] = ea2
    ln = _ln(ea2, g_ref[...], b_ref[...])
    ealn_ref[...] = ln
    eawn_ref[...] = jnp.dot(ln, w1e_ref[...], preferred_element_type=jnp.float32) + b1_ref[...]


def _edge_last_body(ga_ref, gb_ref, eaw_ref, ea_ref, ew2_ref, eb2_ref, ea2_ref):
    hid = jnp.maximum(ga_ref[...] + gb_ref[...] + eaw_ref[...], 0.0)
    ea2_ref[...] = (ea_ref[...]
                    + jnp.dot(hid, ew2_ref[...], preferred_element_type=jnp.float32)
                    + eb2_ref[...])


def _node_update(h, agg, nw1h, nw1a, nb1, nw2, nb2, g, b):
    pre = (jnp.dot(h, nw1h, preferred_element_type=jnp.float32)
           + jnp.dot(agg, nw1a, preferred_element_type=jnp.float32) + nb1)
    h2 = h + jnp.dot(jnp.maximum(pre, 0.0), nw2, preferred_element_type=jnp.float32) + nb2
    return _ln(h2, g, b)


def _agg_from_parts(parts_ref, cntp_ref):
    s = parts_ref[0:_N_NODES] + parts_ref[_ACC:_ACC + _N_NODES]
    cnt = cntp_ref[0:_N_NODES, 0:1] + cntp_ref[_ACC:_ACC + _N_NODES, 0:1]
    return s / jnp.clip(cnt, 1.0, None)


def _node_mid_body(h_ref, parts_ref, cntp_ref,
                   nw1h_ref, nw1a_ref, nb1_ref, nw2_ref, nb2_ref, g_ref, b_ref,
                   w1s_ref, w1d_ref,
                   h_out_ref, gs_ref, gd_ref):
    agg = _agg_from_parts(parts_ref, cntp_ref)
    h1 = _node_update(h_ref[...], agg, nw1h_ref[...], nw1a_ref[...], nb1_ref[...],
                      nw2_ref[...], nb2_ref[...], g_ref[...], b_ref[...])
    h_out_ref[...] = h1
    gs_ref[...] = jnp.dot(h1, w1s_ref[...], preferred_element_type=jnp.float32)
    gd_ref[...] = jnp.dot(h1, w1d_ref[...], preferred_element_type=jnp.float32)


def _node_last_body(h_ref, parts_ref, cntp_ref,
                    nw1h_ref, nw1a_ref, nb1_ref, nw2_ref, nb2_ref, g_ref, b_ref,
                    wm_ref, bm_ref, wl_ref, bl_ref, noise_ref, batch_ref,
                    a_ref, lp_ref):
    agg = _agg_from_parts(parts_ref, cntp_ref)
    h1 = _node_update(h_ref[...], agg, nw1h_ref[...], nw1a_ref[...], nb1_ref[...],
                      nw2_ref[...], nb2_ref[...], g_ref[...], b_ref[...])
    mu = jnp.dot(h1, wm_ref[...], preferred_element_type=jnp.float32) + bm_ref[...]
    t = jnp.tanh(jnp.dot(h1, wl_ref[...], preferred_element_type=jnp.float32) + bl_ref[...])
    log_std = _LOG_STD_MIN + 0.5 * (_LOG_STD_MAX - _LOG_STD_MIN) * (t + 1.0)
    std = jnp.clip(jnp.exp(log_std), 1e-6, 1e6)
    z = mu + std * noise_ref[...]
    a = jnp.tanh(jnp.clip(z, -10.0, 10.0))
    a_ref[...] = a
    log_prob = (-((z - mu) ** 2) / (2.0 * std * std) - jnp.log(std)
                - 0.5 * math.log(2.0 * math.pi))
    logp = log_prob - jnp.log(jnp.clip(1.0 - a * a, 1e-6, None))
    lp = jnp.sum(logp, axis=-1, keepdims=True)
    onehot = (batch_ref[...] ==
              jax.lax.broadcasted_iota(jnp.int32, (1, _NUM_GRAPHS), 1)
              ).astype(jnp.float32)
    lp_ref[...] = jnp.sum(onehot * lp, axis=0, keepdims=True)


def _f32(shape):
    return jax.ShapeDtypeStruct(shape, jnp.float32)


def kernel(x, edge_index, edge_attr, batch, params):
    row = edge_index[0]
    col = edge_index[1]
    p = params
    wx, bx = p['x_proj']
    we, be = p['e_proj']
    blk = p['blocks']
    ew1_0, ew2p_0 = blk[0]['edge_mlp']
    ew1_1, ew2p_1 = blk[1]['edge_mlp']
    w1s0, w1d0, w1e0 = ew1_0[0][:64], ew1_0[0][64:128], ew1_0[0][128:160]
    w1s1, w1d1, w1e1 = ew1_1[0][:64], ew1_1[0][64:128], ew1_1[0][128:160]
    b1_0 = ew1_0[1].reshape(1, -1)
    b1_1 = ew1_1[1].reshape(1, -1)
    ew2_0, eb2_0 = ew2p_0[0], ew2p_0[1].reshape(1, -1)
    ew2_1, eb2_1 = ew2p_1[0], ew2p_1[1].reshape(1, -1)
    noise = jax.random.normal(jax.random.key(42), (_N_NODES, 4), dtype=jnp.float32)

    h0, gs0, gd0 = pl.pallas_call(
        _node0_body,
        out_shape=[_f32((_N_NODES, 64))] * 3,
    )(x, wx, bx.reshape(1, -1), w1s0, w1d0)

    in_specs, out_specs = _edge_specs(
        [16], [32, 64], [we, be.reshape(1, -1), w1e0, b1_0])
    ea0, eaw0 = pl.pallas_call(
        _edge0_body,
        grid=(_EG,), in_specs=in_specs, out_specs=out_specs,
        out_shape=[_f32((_N_EDGES, 32)), _f32((_N_EDGES, 64))],
    )(edge_attr, we, be.reshape(1, -1), w1e0, b1_0)

    ga, gb = _sc_gather(gs0, gd0, row, col)
    w_list = [ew2_0, eb2_0, blk[0]['ln_e'][0].reshape(1, -1),
              blk[0]['ln_e'][1].reshape(1, -1), w1e1, b1_1]
    in_specs, out_specs = _edge_specs([64, 64, 64, 32], [32, 32, 64], w_list)
    ea2_0, ealn_0, eaw1 = pl.pallas_call(
        _edge_mid_body,
        grid=(_EG,), in_specs=in_specs, out_specs=out_specs,
        out_shape=[_f32((_N_EDGES, 32)), _f32((_N_EDGES, 32)), _f32((_N_EDGES, 64))],
    )(ga, gb, eaw0, ea0, *w_list)

    parts0, cnt_parts = _sc_scatter(ea2_0, col, with_cnt=True)
    nw1_0, nb1_0 = blk[0]['node_mlp'][0]
    nw2_0, nb2_0 = blk[0]['node_mlp'][1]
    h1, gs1, gd1 = pl.pallas_call(
        _node_mid_body,
        out_shape=[_f32((_N_NODES, 64))] * 3,
    )(h0, parts0, cnt_parts, nw1_0[:64], nw1_0[64:], nb1_0.reshape(1, -1),
      nw2_0, nb2_0.reshape(1, -1), blk[0]['ln_x'][0].reshape(1, -1),
      blk[0]['ln_x'][1].reshape(1, -1), w1s1, w1d1)

    ga, gb = _sc_gather(gs1, gd1, row, col)
    w_list = [ew2_1, eb2_1]
    in_specs, out_specs = _edge_specs([64, 64, 64, 32], [32], w_list)
    ea2_1, = pl.pallas_call(
        _edge_last_body,
        grid=(_EG,), in_specs=in_specs, out_specs=out_specs,
        out_shape=[_f32((_N_EDGES, 32))],
    )(ga, gb, eaw1, ealn_0, *w_list)

    parts1, _unused = _sc_scatter(ea2_1, col, with_cnt=False)
    nw1_1, nb1_1 = blk[1]['node_mlp'][0]
    nw2_1, nb2_1 = blk[1]['node_mlp'][1]
    a, lp = pl.pallas_call(
        _node_last_body,
        out_shape=[_f32((_N_NODES, 4)), _f32((1, _NUM_GRAPHS))],
    )(h1, parts1, cnt_parts, nw1_1[:64], nw1_1[64:], nb1_1.reshape(1, -1),
      nw2_1, nb2_1.reshape(1, -1), blk[1]['ln_x'][0].reshape(1, -1),
      blk[1]['ln_x'][1].reshape(1, -1),
      p['mu'][0], p['mu'][1].reshape(1, -1),
      p['ls'][0], p['ls'][1].reshape(1, -1),
      noise, batch.reshape(-1, 1))

    return a, lp.reshape(_NUM_GRAPHS, 1)


# speedup vs baseline: 1.2384x; 1.2384x over previous
import functools
import math

import jax
import jax.numpy as jnp
from jax import lax
from jax.experimental import pallas as pl
from jax.experimental.pallas import tpu as pltpu
from jax.experimental.pallas import tpu_sc as plsc

_N_NODES = 10000
_N_EDGES = 320000
_NUM_GRAPHS = 64
_LOG_STD_MIN = -20.0
_LOG_STD_MAX = 2.0

_EC = 2560
_EG = _N_EDGES // _EC

_NC = 2
_NS = 16
_NW = _NC * _NS
_EPW = _N_EDGES // _NW
_CH = 80
_GRP = 5
_GSZ = _CH * _GRP
_NGRP = _EPW // _GSZ
_ACC = 10240
_TPS = _ACC // _NS

_SC_MESH = plsc.VectorSubcoreMesh(
    core_axis_name="c", subcore_axis_name="s", num_cores=_NC, num_subcores=_NS)
_SC_PARAMS = pltpu.CompilerParams(use_tc_tiling_on_sc=False)


def _sc_gather_body(t_hbm, row_hbm, col_hbm, ga_hbm, gb_hbm,
                    idx_a, idx_b, buf_a, buf_b, sem):
    wid = lax.axis_index("s") * _NC + lax.axis_index("c")
    base = wid * _EPW

    def grp(o, carry):
        gbase = pl.multiple_of(base + o * _GSZ, _GSZ)
        pltpu.sync_copy(row_hbm.at[pl.ds(gbase, _GSZ)], idx_a)
        pltpu.sync_copy(col_hbm.at[pl.ds(gbase, _GSZ)], idx_b)
        hs = []
        for j in range(_GRP):
            off = j * _CH
            hs.append(pltpu.async_copy(
                t_hbm.at[idx_a.at[pl.ds(off, _CH)]],
                buf_a.at[pl.ds(off, _CH)], sem))
            hs.append(pltpu.async_copy(
                t_hbm.at[idx_b.at[pl.ds(off, _CH)]],
                buf_b.at[pl.ds(off, _CH)], sem))
        for h in hs:
            h.wait()
        pltpu.sync_copy(buf_a, ga_hbm.at[pl.ds(gbase, _GSZ)])
        pltpu.sync_copy(buf_b, gb_hbm.at[pl.ds(gbase, _GSZ)])
        return carry

    lax.fori_loop(0, _NGRP, grp, 0)


def _sc_gather(t, row, col):
    return pl.kernel(
        _sc_gather_body,
        out_type=[_f32((_N_EDGES, 128)), _f32((_N_EDGES, 128))],
        mesh=_SC_MESH,
        scratch_types=[
            pltpu.VMEM((_GSZ,), jnp.int32),
            pltpu.VMEM((_GSZ,), jnp.int32),
            pltpu.VMEM((_GSZ, 128), jnp.float32),
            pltpu.VMEM((_GSZ, 128), jnp.float32),
            pltpu.SemaphoreType.DMA,
        ],
        compiler_params=_SC_PARAMS,
    )(t, row, col)


def _sc_scatter_body(with_cnt, ea2_hbm, col_hbm, z32_hbm, z16_hbm, ones_hbm,
                     out_hbm, cnt_hbm,
                     idx2, val_buf, stage32, stage16, ones_v,
                     acc, acc_cnt, sem, sem2):
    cid = lax.axis_index("c")
    sid = lax.axis_index("s")
    wid = sid * _NC + cid
    base = wid * _EPW
    trow = pl.multiple_of(sid * _TPS, _TPS)

    pltpu.sync_copy(z32_hbm.at[pl.ds(trow, _TPS)], stage32)
    pltpu.sync_copy(stage32, acc.at[pl.ds(trow, _TPS)])
    if with_cnt:
        pltpu.sync_copy(z16_hbm.at[pl.ds(trow, _TPS)], stage16)
        pltpu.sync_copy(stage16, acc_cnt.at[pl.ds(trow, _TPS)])
        pltpu.sync_copy(ones_hbm, ones_v)
    plsc.subcore_barrier()

    def grp(o, carry):
        gbase = pl.multiple_of(base + o * _GSZ, _GSZ)
        hs = [pltpu.async_copy(ea2_hbm.at[pl.ds(gbase, _GSZ)], val_buf, sem)]
        for j in range(_GRP):
            hs.append(pltpu.async_copy(
                col_hbm.at[pl.ds(gbase + j * _CH, _CH)], idx2.at[j], sem))
        for h in hs:
            h.wait()
        hs = []
        for j in range(_GRP):
            hs.append(pltpu.async_copy(
                val_buf.at[pl.ds(j * _CH, _CH)], acc.at[idx2.at[j]], sem2,
                add=True))
            if with_cnt:
                hs.append(pltpu.async_copy(
                    ones_v, acc_cnt.at[idx2.at[j]], sem2, add=True))
        for h in hs:
            h.wait()
        return carry

    lax.fori_loop(0, _NGRP, grp, 0)
    plsc.subcore_barrier()

    orow = pl.multiple_of(cid * _ACC + sid * _TPS, _TPS)
    pltpu.sync_copy(acc.at[pl.ds(trow, _TPS)], stage32)
    pltpu.sync_copy(stage32, out_hbm.at[pl.ds(orow, _TPS)])
    if with_cnt:
        pltpu.sync_copy(acc_cnt.at[pl.ds(trow, _TPS)], stage16)
        pltpu.sync_copy(stage16, cnt_hbm.at[pl.ds(orow, _TPS)])


def _sc_scatter(ea2, col, with_cnt):
    z32 = jnp.zeros((_ACC, 32), jnp.float32)
    z16 = jnp.zeros((_ACC, 16), jnp.float32)
    ones = jnp.ones((_CH, 16), jnp.float32)
    out_types = [_f32((_NC * _ACC, 32)), _f32((_NC * _ACC, 16))]
    res = pl.kernel(
        functools.partial(_sc_scatter_body, with_cnt),
        out_type=out_types,
        mesh=_SC_MESH,
        scratch_types=[
            pltpu.VMEM((_GRP, _CH), jnp.int32),
            pltpu.VMEM((_GSZ, 32), jnp.float32),
            pltpu.VMEM((_TPS, 32), jnp.float32),
            pltpu.VMEM((_TPS, 16), jnp.float32),
            pltpu.VMEM((_CH, 16), jnp.float32),
            pltpu.VMEM_SHARED((_ACC, 32), jnp.float32),
            pltpu.VMEM_SHARED((_ACC, 16), jnp.float32),
            pltpu.SemaphoreType.DMA,
            pltpu.SemaphoreType.DMA,
        ],
        compiler_params=_SC_PARAMS,
    )(ea2, col, z32, z16, ones)
    return res


def _edge_specs(dims_in, dims_out, weights):
    in_specs = [pl.BlockSpec((_EC, d), lambda i: (i, 0)) for d in dims_in]
    in_specs += [pl.BlockSpec(w.shape, lambda i: (0, 0)) for w in weights]
    out_specs = [pl.BlockSpec((_EC, d), lambda i: (i, 0)) for d in dims_out]
    return in_specs, out_specs


def _ln(v, g, b):
    m = jnp.mean(v, axis=-1, keepdims=True)
    c = v - m
    var = jnp.mean(c * c, axis=-1, keepdims=True)
    return c * jax.lax.rsqrt(var + 1e-5) * g + b


def _node0_body(x_ref, wx_ref, bx_ref, w1sd_ref, h_ref, t_ref):
    h = jnp.maximum(
        jnp.dot(x_ref[...], wx_ref[...], preferred_element_type=jnp.float32)
        + bx_ref[...], 0.0)
    h_ref[...] = h
    t_ref[...] = jnp.dot(h, w1sd_ref[...], preferred_element_type=jnp.float32)


def _edge0_body(eattr_ref, we_ref, be_ref, ea_ref):
    ea_ref[...] = jnp.maximum(
        jnp.dot(eattr_ref[...], we_ref[...], preferred_element_type=jnp.float32)
        + be_ref[...], 0.0)


def _edge_mid_body(ga_ref, gb_ref, ea_ref,
                   w1e_ref, b1_ref, ew2_ref, eb2_ref,
                   ea2_ref):
    ea = ea_ref[...]
    eaw = jnp.dot(ea, w1e_ref[...], preferred_element_type=jnp.float32) + b1_ref[...]
    hid = jnp.maximum(ga_ref[:, :64] + gb_ref[:, 64:] + eaw, 0.0)
    ea2_ref[...] = (ea
                    + jnp.dot(hid, ew2_ref[...], preferred_element_type=jnp.float32)
                    + eb2_ref[...])


def _edge_last_body(ga_ref, gb_ref, ea2p_ref,
                    g_ref, b_ref, w1e_ref, b1_ref, ew2_ref, eb2_ref,
                    ea2_ref):
    ea = _ln(ea2p_ref[...], g_ref[...], b_ref[...])
    eaw = jnp.dot(ea, w1e_ref[...], preferred_element_type=jnp.float32) + b1_ref[...]
    hid = jnp.maximum(ga_ref[:, :64] + gb_ref[:, 64:] + eaw, 0.0)
    ea2_ref[...] = (ea
                    + jnp.dot(hid, ew2_ref[...], preferred_element_type=jnp.float32)
                    + eb2_ref[...])


def _node_update(h, agg, nw1h, nw1a, nb1, nw2, nb2, g, b):
    pre = (jnp.dot(h, nw1h, preferred_element_type=jnp.float32)
           + jnp.dot(agg, nw1a, preferred_element_type=jnp.float32) + nb1)
    h2 = h + jnp.dot(jnp.maximum(pre, 0.0), nw2, preferred_element_type=jnp.float32) + nb2
    return _ln(h2, g, b)


def _agg_from_parts(parts_ref, cntp_ref):
    s = parts_ref[0:_N_NODES] + parts_ref[_ACC:_ACC + _N_NODES]
    cnt = cntp_ref[0:_N_NODES, 0:1] + cntp_ref[_ACC:_ACC + _N_NODES, 0:1]
    return s / jnp.clip(cnt, 1.0, None)


def _node_mid_body(h_ref, parts_ref, cntp_ref,
                   nw1h_ref, nw1a_ref, nb1_ref, nw2_ref, nb2_ref, g_ref, b_ref,
                   w1sd_ref,
                   h_out_ref, t_ref):
    agg = _agg_from_parts(parts_ref, cntp_ref)
    h1 = _node_update(h_ref[...], agg, nw1h_ref[...], nw1a_ref[...], nb1_ref[...],
                      nw2_ref[...], nb2_ref[...], g_ref[...], b_ref[...])
    h_out_ref[...] = h1
    t_ref[...] = jnp.dot(h1, w1sd_ref[...], preferred_element_type=jnp.float32)


def _node_last_body(h_ref, parts_ref, cntp_ref,
                    nw1h_ref, nw1a_ref, nb1_ref, nw2_ref, nb2_ref, g_ref, b_ref,
                    wm_ref, bm_ref, wl_ref, bl_ref, noise_ref, batch_ref,
                    a_ref, lp_ref):
    agg = _agg_from_parts(parts_ref, cntp_ref)
    h1 = _node_update(h_ref[...], agg, nw1h_ref[...], nw1a_ref[...], nb1_ref[...],
                      nw2_ref[...], nb2_ref[...], g_ref[...], b_ref[...])
    mu = jnp.dot(h1, wm_ref[...], preferred_element_type=jnp.float32) + bm_ref[...]
    t = jnp.tanh(jnp.dot(h1, wl_ref[...], preferred_element_type=jnp.float32) + bl_ref[...])
    log_std = _LOG_STD_MIN + 0.5 * (_LOG_STD_MAX - _LOG_STD_MIN) * (t + 1.0)
    std = jnp.clip(jnp.exp(log_std), 1e-6, 1e6)
    z = mu + std * noise_ref[...]
    a = jnp.tanh(jnp.clip(z, -10.0, 10.0))
    a_ref[...] = a
    log_prob = (-((z - mu) ** 2) / (2.0 * std * std) - jnp.log(std)
                - 0.5 * math.log(2.0 * math.pi))
    logp = log_prob - jnp.log(jnp.clip(1.0 - a * a, 1e-6, None))
    lp = jnp.sum(logp, axis=-1, keepdims=True)
    onehot = (batch_ref[...] ==
              jax.lax.broadcasted_iota(jnp.int32, (1, _NUM_GRAPHS), 1)
              ).astype(jnp.float32)
    lp_ref[...] = jnp.sum(onehot * lp, axis=0, keepdims=True)


def _f32(shape):
    return jax.ShapeDtypeStruct(shape, jnp.float32)


def kernel(x, edge_index, edge_attr, batch, params):
    row = edge_index[0]
    col = edge_index[1]
    p = params
    wx, bx = p['x_proj']
    we, be = p['e_proj']
    blk = p['blocks']
    ew1_0, ew2p_0 = blk[0]['edge_mlp']
    ew1_1, ew2p_1 = blk[1]['edge_mlp']
    w1sd0 = jnp.concatenate([ew1_0[0][:64], ew1_0[0][64:128]], axis=1)
    w1sd1 = jnp.concatenate([ew1_1[0][:64], ew1_1[0][64:128]], axis=1)
    w1e0 = ew1_0[0][128:160]
    w1e1 = ew1_1[0][128:160]
    b1_0 = ew1_0[1].reshape(1, -1)
    b1_1 = ew1_1[1].reshape(1, -1)
    ew2_0, eb2_0 = ew2p_0[0], ew2p_0[1].reshape(1, -1)
    ew2_1, eb2_1 = ew2p_1[0], ew2p_1[1].reshape(1, -1)
    noise = jax.random.normal(jax.random.key(42), (_N_NODES, 4), dtype=jnp.float32)

    h0, t0 = pl.pallas_call(
        _node0_body,
        out_shape=[_f32((_N_NODES, 64)), _f32((_N_NODES, 128))],
    )(x, wx, bx.reshape(1, -1), w1sd0)

    in_specs, out_specs = _edge_specs([16], [32], [we, be.reshape(1, -1)])
    ea0, = pl.pallas_call(
        _edge0_body,
        grid=(_EG,), in_specs=in_specs, out_specs=out_specs,
        out_shape=[_f32((_N_EDGES, 32))],
    )(edge_attr, we, be.reshape(1, -1))

    def _half_specs(weights):
        in_specs = [pl.BlockSpec((_EC, 128), lambda i: (i, 0)),
                    pl.BlockSpec((_EC, 128), lambda i: (i, 0)),
                    pl.BlockSpec((_EC, 32), lambda i: (i, 0))]
        in_specs += [pl.BlockSpec(w.shape, lambda i: (0, 0)) for w in weights]
        out_specs = [pl.BlockSpec((_EC, 32), lambda i: (i, 0))]
        return in_specs, out_specs

    ga, gb = _sc_gather(t0, row, col)
    w_list = [w1e0, b1_0, ew2_0, eb2_0]
    in_specs, out_specs = _half_specs(w_list)
    ea2_0, = pl.pallas_call(
        _edge_mid_body,
        grid=(_EG,), in_specs=in_specs, out_specs=out_specs,
        out_shape=[_f32((_N_EDGES, 32))],
    )(ga, gb, ea0, *w_list)

    parts0, cnt_parts = _sc_scatter(ea2_0, col, with_cnt=True)
    nw1_0, nb1_0 = blk[0]['node_mlp'][0]
    nw2_0, nb2_0 = blk[0]['node_mlp'][1]
    h1, t1 = pl.pallas_call(
        _node_mid_body,
        out_shape=[_f32((_N_NODES, 64)), _f32((_N_NODES, 128))],
    )(h0, parts0, cnt_parts, nw1_0[:64], nw1_0[64:], nb1_0.reshape(1, -1),
      nw2_0, nb2_0.reshape(1, -1), blk[0]['ln_x'][0].reshape(1, -1),
      blk[0]['ln_x'][1].reshape(1, -1), w1sd1)

    ga, gb = _sc_gather(t1, row, col)
    w_list = [blk[0]['ln_e'][0].reshape(1, -1), blk[0]['ln_e'][1].reshape(1, -1),
              w1e1, b1_1, ew2_1, eb2_1]
    in_specs, out_specs = _half_specs(w_list)
    ea2_1, = pl.pallas_call(
        _edge_last_body,
        grid=(_EG,), in_specs=in_specs, out_specs=out_specs,
        out_shape=[_f32((_N_EDGES, 32))],
    )(ga, gb, ea2_0, *w_list)

    parts1, _unused = _sc_scatter(ea2_1, col, with_cnt=False)
    nw1_1, nb1_1 = blk[1]['node_mlp'][0]
    nw2_1, nb2_1 = blk[1]['node_mlp'][1]
    a, lp = pl.pallas_call(
        _node_last_body,
        out_shape=[_f32((_N_NODES, 4)), _f32((1, _NUM_GRAPHS))],
    )(h1, parts1, cnt_parts, nw1_1[:64], nw1_1[64:], nb1_1.reshape(1, -1),
      nw2_1, nb2_1.reshape(1, -1), blk[1]['ln_x'][0].reshape(1, -1),
      blk[1]['ln_x'][1].reshape(1, -1),
      p['mu'][0], p['mu'][1].reshape(1, -1),
      p['ls'][0], p['ls'][1].reshape(1, -1),
      noise, batch.reshape(-1, 1))

    return a, lp.reshape(_NUM_GRAPHS, 1)


# speedup vs baseline: 1.5827x; 1.2781x over previous
import functools
import math

import jax
import jax.numpy as jnp
from jax import lax
from jax.experimental import pallas as pl
from jax.experimental.pallas import tpu as pltpu
from jax.experimental.pallas import tpu_sc as plsc

_N_NODES = 10000
_N_EDGES = 320000
_NUM_GRAPHS = 64
_LOG_STD_MIN = -20.0
_LOG_STD_MAX = 2.0

_EC = 2560
_EG = _N_EDGES // _EC

_NC = 2
_NS = 16
_NW = _NC * _NS
_EPW = _N_EDGES // _NW
_CH = 80
_GRP = 5
_GSZ = _CH * _GRP
_NGRP = _EPW // _GSZ
_ACC = 10240
_TPS = _ACC // _NS

_SC_MESH = plsc.VectorSubcoreMesh(
    core_axis_name="c", subcore_axis_name="s", num_cores=_NC, num_subcores=_NS)
_SC_PARAMS = pltpu.CompilerParams(use_tc_tiling_on_sc=False)


def _sc_gather_body(gs_hbm, gd_hbm, row_hbm, col_hbm, s_hbm,
                    idx_a, idx_b, buf_a, buf_b, sbuf, sem):
    wid = lax.axis_index("s") * _NC + lax.axis_index("c")
    base = wid * _EPW

    def grp(o, carry):
        gbase = pl.multiple_of(base + o * _GSZ, _GSZ)
        pltpu.sync_copy(row_hbm.at[pl.ds(gbase, _GSZ)], idx_a)
        pltpu.sync_copy(col_hbm.at[pl.ds(gbase, _GSZ)], idx_b)
        hs = []
        for j in range(_GRP):
            off = j * _CH
            hs.append(pltpu.async_copy(
                gs_hbm.at[idx_a.at[pl.ds(off, _CH)]],
                buf_a.at[pl.ds(off, _CH)], sem))
            hs.append(pltpu.async_copy(
                gd_hbm.at[idx_b.at[pl.ds(off, _CH)]],
                buf_b.at[pl.ds(off, _CH)], sem))
        for h in hs:
            h.wait()

        def add4(r4, carry2):
            r = pl.multiple_of(r4 * 4, 4)
            for u in range(4):
                for k in range(4):
                    sbuf[pl.ds((r + u) * 64 + k * 16, 16)] = (
                        buf_a[r + u, pl.ds(k * 16, 16)]
                        + buf_b[r + u, pl.ds(k * 16, 16)])
            return carry2

        lax.fori_loop(0, _GSZ // 4, add4, 0)
        pltpu.sync_copy(sbuf, s_hbm.at[pl.ds(gbase * 64, _GSZ * 64)])
        return carry

    lax.fori_loop(0, _NGRP, grp, 0)


def _sc_gather(gs, gd, row, col):
    s_flat, = pl.kernel(
        _sc_gather_body,
        out_type=[_f32((_N_EDGES * 64,))],
        mesh=_SC_MESH,
        scratch_types=[
            pltpu.VMEM((_GSZ,), jnp.int32),
            pltpu.VMEM((_GSZ,), jnp.int32),
            pltpu.VMEM((_GSZ, 64), jnp.float32),
            pltpu.VMEM((_GSZ, 64), jnp.float32),
            pltpu.VMEM((_GSZ * 64,), jnp.float32),
            pltpu.SemaphoreType.DMA,
        ],
        compiler_params=_SC_PARAMS,
    )(gs, gd, row, col)
    return s_flat.reshape(_N_EDGES // 2, 128)


def _sc_scatter_body(with_cnt, ea2_hbm, col_hbm, z32_hbm, z16_hbm, ones_hbm,
                     out_hbm, cnt_hbm,
                     idx2, val_buf, stage32, stage16, ones_v,
                     acc, acc_cnt, sem, sem2):
    cid = lax.axis_index("c")
    sid = lax.axis_index("s")
    wid = sid * _NC + cid
    base = wid * _EPW
    trow = pl.multiple_of(sid * _TPS, _TPS)

    pltpu.sync_copy(z32_hbm.at[pl.ds(trow, _TPS)], stage32)
    pltpu.sync_copy(stage32, acc.at[pl.ds(trow, _TPS)])
    if with_cnt:
        pltpu.sync_copy(z16_hbm.at[pl.ds(trow, _TPS)], stage16)
        pltpu.sync_copy(stage16, acc_cnt.at[pl.ds(trow, _TPS)])
        pltpu.sync_copy(ones_hbm, ones_v)
    plsc.subcore_barrier()

    def grp(o, carry):
        gbase = pl.multiple_of(base + o * _GSZ, _GSZ)
        hs = [pltpu.async_copy(ea2_hbm.at[pl.ds(gbase, _GSZ)], val_buf, sem)]
        for j in range(_GRP):
            hs.append(pltpu.async_copy(
                col_hbm.at[pl.ds(gbase + j * _CH, _CH)], idx2.at[j], sem))
        for h in hs:
            h.wait()
        hs = []
        for j in range(_GRP):
            hs.append(pltpu.async_copy(
                val_buf.at[pl.ds(j * _CH, _CH)], acc.at[idx2.at[j]], sem2,
                add=True))
            if with_cnt:
                hs.append(pltpu.async_copy(
                    ones_v, acc_cnt.at[idx2.at[j]], sem2, add=True))
        for h in hs:
            h.wait()
        return carry

    lax.fori_loop(0, _NGRP, grp, 0)
    plsc.subcore_barrier()

    orow = pl.multiple_of(cid * _ACC + sid * _TPS, _TPS)
    pltpu.sync_copy(acc.at[pl.ds(trow, _TPS)], stage32)
    pltpu.sync_copy(stage32, out_hbm.at[pl.ds(orow, _TPS)])
    if with_cnt:
        pltpu.sync_copy(acc_cnt.at[pl.ds(trow, _TPS)], stage16)
        pltpu.sync_copy(stage16, cnt_hbm.at[pl.ds(orow, _TPS)])


def _sc_scatter(ea2, col, with_cnt):
    z32 = jnp.zeros((_ACC, 32), jnp.float32)
    z16 = jnp.zeros((_ACC, 16), jnp.float32)
    ones = jnp.ones((_CH, 16), jnp.float32)
    out_types = [_f32((_NC * _ACC, 32)), _f32((_NC * _ACC, 16))]
    res = pl.kernel(
        functools.partial(_sc_scatter_body, with_cnt),
        out_type=out_types,
        mesh=_SC_MESH,
        scratch_types=[
            pltpu.VMEM((_GRP, _CH), jnp.int32),
            pltpu.VMEM((_GSZ, 32), jnp.float32),
            pltpu.VMEM((_TPS, 32), jnp.float32),
            pltpu.VMEM((_TPS, 16), jnp.float32),
            pltpu.VMEM((_CH, 16), jnp.float32),
            pltpu.VMEM_SHARED((_ACC, 32), jnp.float32),
            pltpu.VMEM_SHARED((_ACC, 16), jnp.float32),
            pltpu.SemaphoreType.DMA,
            pltpu.SemaphoreType.DMA,
        ],
        compiler_params=_SC_PARAMS,
    )(ea2, col, z32, z16, ones)
    return res


def _edge_specs(dims_in, dims_out, weights):
    in_specs = [pl.BlockSpec((_EC, d), lambda i: (i, 0)) for d in dims_in]
    in_specs += [pl.BlockSpec(w.shape, lambda i: (0, 0)) for w in weights]
    out_specs = [pl.BlockSpec((_EC, d), lambda i: (i, 0)) for d in dims_out]
    return in_specs, out_specs


def _ln(v, g, b):
    m = jnp.mean(v, axis=-1, keepdims=True)
    c = v - m
    var = jnp.mean(c * c, axis=-1, keepdims=True)
    return c * jax.lax.rsqrt(var + 1e-5) * g + b


def _node0_body(x_ref, wx_ref, bx_ref, w1s_ref, w1d_ref, h_ref, gs_ref, gd_ref):
    h = jnp.maximum(
        jnp.dot(x_ref[...], wx_ref[...], preferred_element_type=jnp.float32)
        + bx_ref[...], 0.0)
    h_ref[...] = h
    gs_ref[...] = jnp.dot(h, w1s_ref[...], preferred_element_type=jnp.float32)
    gd_ref[...] = jnp.dot(h, w1d_ref[...], preferred_element_type=jnp.float32)


def _edge_mid_body(s_ref, eattr_ref,
                   we_ref, be_ref, w1e_ref, b1_ref, ew2_ref, eb2_ref,
                   ea2_ref):
    ea = jnp.maximum(
        jnp.dot(eattr_ref[...], we_ref[...], preferred_element_type=jnp.float32)
        + be_ref[...], 0.0)
    hid = jnp.maximum(
        s_ref[...]
        + jnp.dot(ea, w1e_ref[...], preferred_element_type=jnp.float32)
        + b1_ref[...], 0.0)
    ea2_ref[...] = (ea
                    + jnp.dot(hid, ew2_ref[...], preferred_element_type=jnp.float32)
                    + eb2_ref[...])


def _edge_last_body(s_ref, ea2p_ref,
                    g_ref, b_ref, w1e_ref, b1_ref, ew2_ref, eb2_ref,
                    ea2_ref):
    x = ea2p_ref[...]
    ea = jnp.concatenate(
        [_ln(x[:, :32], g_ref[...], b_ref[...]),
         _ln(x[:, 32:], g_ref[...], b_ref[...])], axis=1)
    hid = jnp.maximum(
        s_ref[...]
        + jnp.dot(ea, w1e_ref[...], preferred_element_type=jnp.float32)
        + b1_ref[...], 0.0)
    ea2_ref[...] = (ea
                    + jnp.dot(hid, ew2_ref[...], preferred_element_type=jnp.float32)
                    + eb2_ref[...])


def _node_update(h, agg, nw1h, nw1a, nb1, nw2, nb2, g, b):
    pre = (jnp.dot(h, nw1h, preferred_element_type=jnp.float32)
           + jnp.dot(agg, nw1a, preferred_element_type=jnp.float32) + nb1)
    h2 = h + jnp.dot(jnp.maximum(pre, 0.0), nw2, preferred_element_type=jnp.float32) + nb2
    return _ln(h2, g, b)


def _agg_from_parts(parts_ref, cntp_ref):
    s = parts_ref[0:_N_NODES] + parts_ref[_ACC:_ACC + _N_NODES]
    cnt = cntp_ref[0:_N_NODES, 0:1] + cntp_ref[_ACC:_ACC + _N_NODES, 0:1]
    return s / jnp.clip(cnt, 1.0, None)


def _node_mid_body(h_ref, parts_ref, cntp_ref,
                   nw1h_ref, nw1a_ref, nb1_ref, nw2_ref, nb2_ref, g_ref, b_ref,
                   w1s_ref, w1d_ref,
                   h_out_ref, gs_ref, gd_ref):
    agg = _agg_from_parts(parts_ref, cntp_ref)
    h1 = _node_update(h_ref[...], agg, nw1h_ref[...], nw1a_ref[...], nb1_ref[...],
                      nw2_ref[...], nb2_ref[...], g_ref[...], b_ref[...])
    h_out_ref[...] = h1
    gs_ref[...] = jnp.dot(h1, w1s_ref[...], preferred_element_type=jnp.float32)
    gd_ref[...] = jnp.dot(h1, w1d_ref[...], preferred_element_type=jnp.float32)


def _node_last_body(h_ref, parts_ref, cntp_ref,
                    nw1h_ref, nw1a_ref, nb1_ref, nw2_ref, nb2_ref, g_ref, b_ref,
                    wm_ref, bm_ref, wl_ref, bl_ref, noise_ref, batch_ref,
                    a_ref, lp_ref):
    agg = _agg_from_parts(parts_ref, cntp_ref)
    h1 = _node_update(h_ref[...], agg, nw1h_ref[...], nw1a_ref[...], nb1_ref[...],
                      nw2_ref[...], nb2_ref[...], g_ref[...], b_ref[...])
    mu = jnp.dot(h1, wm_ref[...], preferred_element_type=jnp.float32) + bm_ref[...]
    t = jnp.tanh(jnp.dot(h1, wl_ref[...], preferred_element_type=jnp.float32) + bl_ref[...])
    log_std = _LOG_STD_MIN + 0.5 * (_LOG_STD_MAX - _LOG_STD_MIN) * (t + 1.0)
    std = jnp.clip(jnp.exp(log_std), 1e-6, 1e6)
    z = mu + std * noise_ref[...]
    a = jnp.tanh(jnp.clip(z, -10.0, 10.0))
    a_ref[...] = a
    log_prob = (-((z - mu) ** 2) / (2.0 * std * std) - jnp.log(std)
                - 0.5 * math.log(2.0 * math.pi))
    logp = log_prob - jnp.log(jnp.clip(1.0 - a * a, 1e-6, None))
    lp = jnp.sum(logp, axis=-1, keepdims=True)
    onehot = (batch_ref[...] ==
              jax.lax.broadcasted_iota(jnp.int32, (1, _NUM_GRAPHS), 1)
              ).astype(jnp.float32)
    lp_ref[...] = jnp.sum(onehot * lp, axis=0, keepdims=True)


def _f32(shape):
    return jax.ShapeDtypeStruct(shape, jnp.float32)


def kernel(x, edge_index, edge_attr, batch, params):
    row = edge_index[0]
    col = edge_index[1]
    p = params
    wx, bx = p['x_proj']
    we, be = p['e_proj']
    blk = p['blocks']
    ew1_0, ew2p_0 = blk[0]['edge_mlp']
    ew1_1, ew2p_1 = blk[1]['edge_mlp']
    w1s0, w1d0, w1e0 = ew1_0[0][:64], ew1_0[0][64:128], ew1_0[0][128:160]
    w1s1, w1d1, w1e1 = ew1_1[0][:64], ew1_1[0][64:128], ew1_1[0][128:160]

    def _bdiag(w):
        z = jnp.zeros_like(w)
        return jnp.concatenate(
            [jnp.concatenate([w, z], axis=1), jnp.concatenate([z, w], axis=1)],
            axis=0)

    def _tile2(b):
        return jnp.concatenate([b, b]).reshape(1, -1)

    we_d = _bdiag(we)
    be_d = _tile2(be)
    w1e0_d, b1_0d = _bdiag(w1e0), _tile2(ew1_0[1])
    w1e1_d, b1_1d = _bdiag(w1e1), _tile2(ew1_1[1])
    ew2_0d, eb2_0d = _bdiag(ew2p_0[0]), _tile2(ew2p_0[1])
    ew2_1d, eb2_1d = _bdiag(ew2p_1[0]), _tile2(ew2p_1[1])
    noise = jax.random.normal(jax.random.key(42), (_N_NODES, 4), dtype=jnp.float32)

    _EH = _N_EDGES // 2
    _ECH = _EC // 2
    eattr_p2 = edge_attr.reshape(_EH, 32)

    h0, gs0, gd0 = pl.pallas_call(
        _node0_body,
        out_shape=[_f32((_N_NODES, 64))] * 3,
    )(x, wx, bx.reshape(1, -1), w1s0, w1d0)

    def _p2_specs(in_dims, weights):
        in_specs = [pl.BlockSpec((_ECH, d), lambda i: (i, 0)) for d in in_dims]
        in_specs += [pl.BlockSpec(w.shape, lambda i: (0, 0)) for w in weights]
        out_specs = [pl.BlockSpec((_ECH, 64), lambda i: (i, 0))]
        return in_specs, out_specs

    s0 = _sc_gather(gs0, gd0, row, col)
    w_list = [we_d, be_d, w1e0_d, b1_0d, ew2_0d, eb2_0d]
    in_specs, out_specs = _p2_specs([128, 32], w_list)
    ea2_0, = pl.pallas_call(
        _edge_mid_body,
        grid=(_EG,), in_specs=in_specs, out_specs=out_specs,
        out_shape=[_f32((_EH, 64))],
    )(s0, eattr_p2, *w_list)

    parts0, cnt_parts = _sc_scatter(ea2_0.reshape(_N_EDGES, 32), col, with_cnt=True)
    nw1_0, nb1_0 = blk[0]['node_mlp'][0]
    nw2_0, nb2_0 = blk[0]['node_mlp'][1]
    h1, gs1, gd1 = pl.pallas_call(
        _node_mid_body,
        out_shape=[_f32((_N_NODES, 64))] * 3,
    )(h0, parts0, cnt_parts, nw1_0[:64], nw1_0[64:], nb1_0.reshape(1, -1),
      nw2_0, nb2_0.reshape(1, -1), blk[0]['ln_x'][0].reshape(1, -1),
      blk[0]['ln_x'][1].reshape(1, -1), w1s1, w1d1)

    s1 = _sc_gather(gs1, gd1, row, col)
    w_list = [blk[0]['ln_e'][0].reshape(1, -1), blk[0]['ln_e'][1].reshape(1, -1),
              w1e1_d, b1_1d, ew2_1d, eb2_1d]
    in_specs, out_specs = _p2_specs([128, 64], w_list)
    ea2_1, = pl.pallas_call(
        _edge_last_body,
        grid=(_EG,), in_specs=in_specs, out_specs=out_specs,
        out_shape=[_f32((_EH, 64))],
    )(s1, ea2_0, *w_list)

    ea2_1 = ea2_1.reshape(_N_EDGES, 32)

    parts1, _unused = _sc_scatter(ea2_1, col, with_cnt=False)
    nw1_1, nb1_1 = blk[1]['node_mlp'][0]
    nw2_1, nb2_1 = blk[1]['node_mlp'][1]
    a, lp = pl.pallas_call(
        _node_last_body,
        out_shape=[_f32((_N_NODES, 4)), _f32((1, _NUM_GRAPHS))],
    )(h1, parts1, cnt_parts, nw1_1[:64], nw1_1[64:], nb1_1.reshape(1, -1),
      nw2_1, nb2_1.reshape(1, -1), blk[1]['ln_x'][0].reshape(1, -1),
      blk[1]['ln_x'][1].reshape(1, -1),
      p['mu'][0], p['mu'][1].reshape(1, -1),
      p['ls'][0], p['ls'][1].reshape(1, -1),
      noise, batch.reshape(-1, 1))

    return a, lp.reshape(_NUM_GRAPHS, 1)
